# baseline (speedup 1.0000x reference)
import functools

import jax
import jax.numpy as jnp
from jax import lax
from jax.experimental import pallas as pl
from jax.experimental.pallas import tpu as pltpu
from jax.experimental.pallas import tpu_sc as plsc


def _scores_body(lt_ref, pres_ref, scores_ref, labels_ref):
    x = lt_ref[...]
    p = pres_ref[...]
    q2 = 1.0 + jnp.exp(-p.T)
    q = q2[:, :, None]
    t = q * jnp.exp(-x) + q
    m = jnp.min(t, axis=0)
    scores_ref[...] = 1.0 / m
    labels_ref[...] = jnp.ones(labels_ref.shape, jnp.int32)


def _sc_boxes(pbt_hbm, scale_hbm, out_hbm, a_v, wh_v, o_v, s_v, sem):
    wid = lax.axis_index("s") * 2 + lax.axis_index("c")
    b = wid // 4
    r = wid - 4 * b
    ai = r & 1
    N = a_v.shape[0]
    pltpu.sync_copy(pbt_hbm.at[b, ai], a_v)
    pltpu.sync_copy(pbt_hbm.at[b, 2 + ai], wh_v)
    pltpu.sync_copy(scale_hbm.at[b, ai], s_v)
    sgn = jnp.where(r < 2, -0.5, 0.5)
    sv = s_v[...]

    def body(i, _):
        sl = pl.ds(i * 16, 16)
        o_v[sl] = (a_v[sl] + sgn * wh_v[sl]) * sv
        return 0

    lax.fori_loop(0, N // 16, body, 0)
    pltpu.sync_copy(o_v, out_hbm.at[b, r])


def kernel(pred_logits, pred_boxes, presence_logit_dec, target_sizes_boxes,
           target_sizes_masks):
    B, N, C = pred_logits.shape
    TILE = 2048
    ntiles = (N + TILE - 1) // TILE

    lt = jnp.transpose(pred_logits, (2, 0, 1))
    pbt = jnp.transpose(pred_boxes, (0, 2, 1))
    img_h = target_sizes_boxes[:, 0].astype(jnp.float32)
    img_w = target_sizes_boxes[:, 1].astype(jnp.float32)
    scale_bc = jnp.broadcast_to(
        jnp.stack([img_w, img_h], axis=1)[:, :, None], (B, 2, 16))

    mesh = plsc.VectorSubcoreMesh(core_axis_name="c", subcore_axis_name="s")
    boxes_t = functools.partial(
        pl.kernel,
        mesh=mesh,
        out_type=jax.ShapeDtypeStruct((B, 4, N), jnp.float32),
        scratch_types=[
            pltpu.VMEM((N,), jnp.float32),
            pltpu.VMEM((N,), jnp.float32),
            pltpu.VMEM((N,), jnp.float32),
            pltpu.VMEM((16,), jnp.float32),
            pltpu.SemaphoreType.DMA,
        ],
    )(_sc_boxes)(pbt, scale_bc)

    scores, labels = pl.pallas_call(
        _scores_body,
        grid=(ntiles,),
        in_specs=[
            pl.BlockSpec((C, B, TILE), lambda i: (0, 0, i)),
            pl.BlockSpec((B, C), lambda i: (0, 0)),
        ],
        out_specs=[
            pl.BlockSpec((B, TILE), lambda i: (0, i)),
            pl.BlockSpec((B, TILE), lambda i: (0, i)),
        ],
        out_shape=[
            jax.ShapeDtypeStruct((B, N), jnp.float32),
            jax.ShapeDtypeStruct((B, N), jnp.int32),
        ],
    )(lt, presence_logit_dec)

    boxes = jnp.transpose(boxes_t, (0, 2, 1))
    return scores, labels, boxes


# speedup vs baseline: 1.4849x; 1.4849x over previous
import jax
import jax.numpy as jnp
from jax.experimental import pallas as pl
from jax.experimental.pallas import tpu as pltpu


def _body(lt_ref, pres_ref, pbt_ref, ts_ref, scores_ref, obox_ref, labels_ref):
    x = lt_ref[...]
    p = pres_ref[...]
    q2 = 1.0 + jnp.exp(-p.T)
    q = q2[:, :, None]
    t = q * jnp.exp(-x) + q
    m = jnp.min(t, axis=0)
    scores_ref[...] = 1.0 / m

    bx = pbt_ref[...]
    ts = ts_ref[...].astype(jnp.float32)
    h = ts[:, 0:1, :]
    w = ts[:, 1:2, :]
    scale = jnp.concatenate([w, h, w, h], axis=1)
    rolled = pltpu.roll(bx, 2, 1)
    row = jax.lax.broadcasted_iota(jnp.int32, bx.shape, 1)
    first = row < 2
    a = jnp.where(first, bx, rolled)
    wh = jnp.where(first, rolled, bx)
    sign = jnp.where(first, -0.5, 0.5)
    obox_ref[...] = (a + sign * wh) * scale

    labels_ref[...] = jnp.ones(labels_ref.shape, jnp.int32)


def kernel(pred_logits, pred_boxes, presence_logit_dec, target_sizes_boxes,
           target_sizes_masks):
    B, N, C = pred_logits.shape
    TILE = 1792
    ntiles = (N + TILE - 1) // TILE

    lt = jnp.transpose(pred_logits, (2, 0, 1))
    pbt = jnp.transpose(pred_boxes, (0, 2, 1))
    ts3 = target_sizes_boxes.reshape(B, 2, 1)

    scores, boxes_t, labels = pl.pallas_call(
        _body,
        grid=(ntiles,),
        in_specs=[
            pl.BlockSpec((C, B, TILE), lambda i: (0, 0, i)),
            pl.BlockSpec((B, C), lambda i: (0, 0)),
            pl.BlockSpec((B, 4, TILE), lambda i: (0, 0, i)),
            pl.BlockSpec((B, 2, 1), lambda i: (0, 0, 0)),
        ],
        out_specs=[
            pl.BlockSpec((B, TILE), lambda i: (0, i)),
            pl.BlockSpec((B, 4, TILE), lambda i: (0, 0, i)),
            pl.BlockSpec((B, TILE), lambda i: (0, i)),
        ],
        out_shape=[
            jax.ShapeDtypeStruct((B, N), jnp.float32),
            jax.ShapeDtypeStruct((B, 4, N), jnp.float32),
            jax.ShapeDtypeStruct((B, N), jnp.int32),
        ],
    )(lt, presence_logit_dec, pbt, ts3)

    boxes = jnp.transpose(boxes_t, (0, 2, 1))
    return scores, labels, boxes


# speedup vs baseline: 1.5702x; 1.0575x over previous
import jax
import jax.numpy as jnp
from jax.experimental import pallas as pl
from jax.experimental.pallas import tpu as pltpu


def _body(lt_ref, pres_ref, pbt_ref, ts_ref, scores_ref, obox_ref, labels_ref):
    x = lt_ref[...]
    p = pres_ref[...]
    q2 = 1.0 + jnp.exp(-p.T)
    q = q2[:, :, None]
    t = q * jnp.exp(-x) + q
    m = jnp.min(t, axis=0)
    scores_ref[...] = 1.0 / m

    bx = pbt_ref[...]
    ts = ts_ref[...].astype(jnp.float32)
    h = ts[:, 0:1, :]
    w = ts[:, 1:2, :]
    scale = jnp.concatenate([w, h, w, h], axis=1)
    rolled = pltpu.roll(bx, 2, 1)
    row = jax.lax.broadcasted_iota(jnp.int32, bx.shape, 1)
    first = row < 2
    a = jnp.where(first, bx, rolled)
    wh = jnp.where(first, rolled, bx)
    sign = jnp.where(first, -0.5, 0.5)
    obox_ref[...] = (a + sign * wh) * scale

    labels_ref[...] = jnp.ones(labels_ref.shape, jnp.int32)


def kernel(pred_logits, pred_boxes, presence_logit_dec, target_sizes_boxes,
           target_sizes_masks):
    B, N, C = pred_logits.shape
    TILE = 2048
    ntiles = (N + TILE - 1) // TILE

    lt = jnp.transpose(pred_logits, (2, 0, 1))
    pbt = jnp.transpose(pred_boxes, (0, 2, 1))
    ts3 = target_sizes_boxes.reshape(B, 2, 1)

    scores, boxes_t, labels = pl.pallas_call(
        _body,
        grid=(ntiles,),
        in_specs=[
            pl.BlockSpec((C, B, TILE), lambda i: (0, 0, i)),
            pl.BlockSpec((B, C), lambda i: (0, 0)),
            pl.BlockSpec((B, 4, TILE), lambda i: (0, 0, i)),
            pl.BlockSpec((B, 2, 1), lambda i: (0, 0, 0)),
        ],
        out_specs=[
            pl.BlockSpec((B, TILE), lambda i: (0, i)),
            pl.BlockSpec((B, 4, TILE), lambda i: (0, 0, i)),
            pl.BlockSpec((B, TILE), lambda i: (0, i)),
        ],
        out_shape=[
            jax.ShapeDtypeStruct((B, N), jnp.float32),
            jax.ShapeDtypeStruct((B, 4, N), jnp.float32),
            jax.ShapeDtypeStruct((B, N), jnp.int32),
        ],
    )(lt, presence_logit_dec, pbt, ts3)

    boxes = jnp.transpose(boxes_t, (0, 2, 1))
    return scores, labels, boxes
